# baseline (speedup 1.0000x reference)
import functools
import jax
import jax.numpy as jnp
from jax.experimental import pallas as pl
from jax.experimental.pallas import tpu as pltpu

_F32 = jnp.float32
_INF = float("inf")


def _fold_mlp(layers):
    out = []
    for layer in layers:
        W = layer["W"]
        if "gamma" in layer:
            W = W * (layer["gamma"] / jnp.sqrt(1.0 + 1e-5))[None, :]
            b = layer["beta"]
            relu = True
        else:
            b = layer["b"]
            relu = False
        out.append((W, b.reshape(1, -1), relu))
    return out


def _apply_tail(h, wb_refs, relus):
    for i, relu in enumerate(relus):
        h = h + wb_refs[2 * i + 1][...]
        if relu:
            h = jnp.maximum(h, 0.0)
        if i + 1 < len(relus):
            h = jnp.dot(h, wb_refs[2 * (i + 1)][...],
                        preferred_element_type=_F32)
    return h


def _fps_body(n_out, px_ref, py_ref, pz_ref, ox_ref, oy_ref, oz_ref, dist_ref):
    px = px_ref[...]
    py = py_ref[...]
    pz = pz_ref[...]
    B, N = px.shape
    iota_n = jax.lax.broadcasted_iota(jnp.int32, (B, N), 1)
    iota_o = jax.lax.broadcasted_iota(jnp.int32, (B, n_out), 1)
    dist_ref[...] = jnp.full((B, N), _INF, _F32)
    ox_ref[...] = jnp.zeros((B, n_out), _F32)
    oy_ref[...] = jnp.zeros((B, n_out), _F32)
    oz_ref[...] = jnp.zeros((B, n_out), _F32)

    def step(t, last):
        oh = iota_n == last
        sx = jnp.sum(jnp.where(oh, px, 0.0), axis=1, keepdims=True)
        sy = jnp.sum(jnp.where(oh, py, 0.0), axis=1, keepdims=True)
        sz = jnp.sum(jnp.where(oh, pz, 0.0), axis=1, keepdims=True)
        rec = iota_o == t
        ox_ref[...] += jnp.where(rec, sx, 0.0)
        oy_ref[...] += jnp.where(rec, sy, 0.0)
        oz_ref[...] += jnp.where(rec, sz, 0.0)
        d2 = (px - sx) ** 2 + (py - sy) ** 2 + (pz - sz) ** 2
        nd = jnp.minimum(dist_ref[...], d2)
        dist_ref[...] = nd
        nxt = jnp.argmax(nd, axis=1).astype(jnp.int32)[:, None]
        return nxt

    jax.lax.fori_loop(0, n_out, step, jnp.zeros((B, 1), jnp.int32))


def _fps_call(px, py, pz, n_out):
    B, N = px.shape
    out_sd = jax.ShapeDtypeStruct((B, n_out), _F32)
    return pl.pallas_call(
        functools.partial(_fps_body, n_out),
        out_shape=(out_sd, out_sd, out_sd),
        scratch_shapes=[pltpu.VMEM((B, N), _F32)],
    )(px, py, pz)


def _sa_body(qb, S, K, r2, relus_l, relus_g, nlayers_l, has_feat, *refs):
    qx = refs[0][0]
    qy = refs[1][0]
    qz = refs[2][0]
    px = refs[3][0]
    py = refs[4][0]
    pz = refs[5][0]
    i = 6
    feat = None
    if has_feat:
        feat = refs[i][0]
        i += 1
    wb = refs[i:i + 2 * (nlayers_l + len(relus_g))]
    o_ref = refs[-2]
    d2_ref = refs[-1]
    wb_l = wb[: 2 * nlayers_l]
    wb_g = wb[2 * nlayers_l:]

    d2 = (qx - px) ** 2 + (qy - py) ** 2 + (qz - pz) ** 2
    d2_ref[...] = jnp.where(d2 <= r2, d2, _INF)
    iota_s = jax.lax.broadcasted_iota(jnp.int32, (qb, S), 1)
    w1 = wb_l[0]
    c_feat = w1.shape[0] - 3
    cmid = wb_l[2 * (nlayers_l - 1)].shape[1]

    def rnd(t, acc):
        d2m = d2_ref[...]
        amin = jnp.argmin(d2m, axis=1).astype(jnp.int32)[:, None]
        oh = iota_s == amin
        m = jnp.sum(jnp.where(oh, d2m, 0.0), axis=1, keepdims=True)
        valid = m < _INF
        d2_ref[...] = jnp.where(oh, _INF, d2m)
        nx = jnp.sum(jnp.where(oh, px, 0.0), axis=1, keepdims=True)
        ny = jnp.sum(jnp.where(oh, py, 0.0), axis=1, keepdims=True)
        nz = jnp.sum(jnp.where(oh, pz, 0.0), axis=1, keepdims=True)
        if has_feat:
            xj = jnp.dot(oh.astype(_F32), feat, preferred_element_type=_F32)
            h = jnp.dot(xj, w1[0:c_feat, :], preferred_element_type=_F32)
        else:
            h = nx * w1[0:1, :] + ny * w1[1:2, :] + nz * w1[2:3, :]
        h = (h + (nx - qx) * w1[c_feat:c_feat + 1, :]
               + (ny - qy) * w1[c_feat + 1:c_feat + 2, :]
               + (nz - qz) * w1[c_feat + 2:c_feat + 3, :])
        h = _apply_tail(h, wb_l, relus_l)
        h = jnp.where(valid, h, -_INF)
        return jnp.maximum(acc, h)

    acc0 = jnp.full((qb, cmid), -_INF, _F32)
    acc = jax.lax.fori_loop(0, K, rnd, acc0)
    hg = jnp.dot(acc, wb_g[0][...], preferred_element_type=_F32)
    o_ref[0] = _apply_tail(hg, wb_g, relus_g)


def _sa_call(qpos, spos, feat, layers_l, layers_g, K, r2, qb):
    B, Nq, _ = qpos[0].shape
    S = spos[0].shape[2]
    relus_l = tuple(r for _, _, r in layers_l)
    relus_g = tuple(r for _, _, r in layers_g)
    cout = layers_g[-1][0].shape[1]
    grid = (B, Nq // qb)
    specs = []
    args = []
    for q in qpos:
        args.append(q)
        specs.append(pl.BlockSpec((1, qb, 1), lambda b, j: (b, j, 0)))
    for s in spos:
        args.append(s)
        specs.append(pl.BlockSpec((1, 1, S), lambda b, j: (b, 0, 0)))
    if feat is not None:
        C = feat.shape[2]
        args.append(feat)
        specs.append(pl.BlockSpec((1, S, C), lambda b, j: (b, 0, 0)))
    wbs = []
    for W, bias, _ in list(layers_l) + list(layers_g):
        wbs.extend([W, bias])
    for w in wbs:
        args.append(w)
        specs.append(pl.BlockSpec(w.shape, lambda b, j: (0, 0)))
    body = functools.partial(_sa_body, qb, S, K, r2, relus_l, relus_g,
                             len(layers_l), feat is not None)
    return pl.pallas_call(
        body,
        grid=grid,
        in_specs=specs,
        out_specs=pl.BlockSpec((1, qb, cout), lambda b, j: (b, j, 0)),
        out_shape=jax.ShapeDtypeStruct((B, Nq, cout), _F32),
        scratch_shapes=[pltpu.VMEM((qb, S), _F32)],
    )(*args)


def _glob_body(relus, x_ref, w1_ref, b1_ref, w2_ref, b2_ref, o_ref):
    x = x_ref[0]
    mx = jnp.max(x, axis=0, keepdims=True)
    h = jnp.dot(mx, w1_ref[...], preferred_element_type=_F32)
    o_ref[0] = _apply_tail(h, (w1_ref, b1_ref, w2_ref, b2_ref), relus)


def _glob_call(x2, layers):
    B, Np, C = x2.shape
    relus = tuple(r for _, _, r in layers)
    cout = layers[-1][0].shape[1]
    (w1, b1, _), (w2, b2, _) = layers
    return pl.pallas_call(
        functools.partial(_glob_body, relus),
        grid=(B,),
        in_specs=[
            pl.BlockSpec((1, Np, C), lambda b: (b, 0, 0)),
            pl.BlockSpec(w1.shape, lambda b: (0, 0)),
            pl.BlockSpec(b1.shape, lambda b: (0, 0)),
            pl.BlockSpec(w2.shape, lambda b: (0, 0)),
            pl.BlockSpec(b2.shape, lambda b: (0, 0)),
        ],
        out_specs=pl.BlockSpec((1, 1, cout), lambda b: (b, 0, 0)),
        out_shape=jax.ShapeDtypeStruct((B, 1, cout), _F32),
    )(x2, w1, b1, w2, b2)


def _interp_body(qb, S, k, relus, nskip, *refs):
    qx = refs[0][0]
    qy = refs[1][0]
    qz = refs[2][0]
    px = refs[3][0]
    py = refs[4][0]
    pz = refs[5][0]
    feat = refs[6][0]
    i = 7
    skip = None
    if nskip > 0:
        skip = refs[i][0]
        i += 1
    wb = refs[i:-1]
    o_ref = refs[-1]

    d2m = (qx - px) ** 2 + (qy - py) ** 2 + (qz - pz) ** 2
    iota_s = jax.lax.broadcasted_iota(jnp.int32, (qb, S), 1)
    C = feat.shape[1]
    num = jnp.zeros((qb, C), _F32)
    den = jnp.zeros((qb, 1), _F32)
    for _ in range(k):
        amin = jnp.argmin(d2m, axis=1).astype(jnp.int32)[:, None]
        oh = iota_s == amin
        m = jnp.sum(jnp.where(oh, d2m, 0.0), axis=1, keepdims=True)
        d2m = jnp.where(oh, _INF, d2m)
        w = 1.0 / jnp.maximum(m, 1e-16)
        ft = jnp.dot(oh.astype(_F32), feat, preferred_element_type=_F32)
        num = num + w * ft
        den = den + w
    up = num / den
    w1 = wb[0]
    h = jnp.dot(up, w1[0:C, :], preferred_element_type=_F32)
    if nskip > 0:
        cs = skip.shape[1]
        h = h + jnp.dot(skip, w1[C:C + cs, :], preferred_element_type=_F32)
    else:
        h = (h + qx * w1[C:C + 1, :] + qy * w1[C + 1:C + 2, :]
               + qz * w1[C + 2:C + 3, :])
    o_ref[0] = _apply_tail(h, wb, relus)


def _interp_call(qpos, spos, feat, skip, layers, k, qb):
    B, Nq, _ = qpos[0].shape
    S = feat.shape[1]
    relus = tuple(r for _, _, r in layers)
    cout = layers[-1][0].shape[1]
    C = feat.shape[2]
    grid = (B, Nq // qb)
    specs = []
    args = []
    for q in qpos:
        args.append(q)
        specs.append(pl.BlockSpec((1, qb, 1), lambda b, j: (b, j, 0)))
    for s in spos:
        args.append(s)
        specs.append(pl.BlockSpec((1, 1, S), lambda b, j: (b, 0, 0)))
    args.append(feat)
    specs.append(pl.BlockSpec((1, S, C), lambda b, j: (b, 0, 0)))
    nskip = 0
    if skip is not None:
        nskip = skip.shape[2]
        args.append(skip)
        specs.append(pl.BlockSpec((1, qb, nskip), lambda b, j: (b, j, 0)))
    for W, bias, _ in layers:
        args.append(W)
        specs.append(pl.BlockSpec(W.shape, lambda b, j: (0, 0)))
        args.append(bias)
        specs.append(pl.BlockSpec(bias.shape, lambda b, j: (0, 0)))
    body = functools.partial(_interp_body, qb, S, k, relus, nskip)
    return pl.pallas_call(
        body,
        grid=grid,
        in_specs=specs,
        out_specs=pl.BlockSpec((1, qb, cout), lambda b, j: (b, j, 0)),
        out_shape=jax.ShapeDtypeStruct((B, Nq, cout), _F32),
    )(*args)


def _encode(pts, params, n1, n2, r1, r2, max_n1, max_n2, k_fp, qb1, qb0):
    B, N, _ = pts.shape
    px = pts[:, :, 0]
    py = pts[:, :, 1]
    pz = pts[:, :, 2]

    sa1_l = _fold_mlp(params["sa1_local"])
    sa1_g = _fold_mlp(params["sa1_global"])
    sa2_l = _fold_mlp(params["sa2_local"])
    sa2_g = _fold_mlp(params["sa2_global"])
    glob = _fold_mlp(params["glob"])
    fp1 = _fold_mlp(params["fp1"])
    fp0 = _fold_mlp(params["fp0"])

    c1x, c1y, c1z = _fps_call(px, py, pz, n1)
    q1 = [c1x[..., None], c1y[..., None], c1z[..., None]]
    s0 = [px[:, None, :], py[:, None, :], pz[:, None, :]]
    x1 = _sa_call(q1, s0, None, sa1_l, sa1_g, max_n1, r1 * r1, qb1)

    c2x, c2y, c2z = _fps_call(c1x, c1y, c1z, n2)
    q2 = [c2x[..., None], c2y[..., None], c2z[..., None]]
    s1 = [c1x[:, None, :], c1y[:, None, :], c1z[:, None, :]]
    x2 = _sa_call(q2, s1, x1, sa2_l, sa2_g, max_n2, r2 * r2, n2)

    g = _glob_call(x2, glob)[:, 0, :]

    s2 = [c2x[:, None, :], c2y[:, None, :], c2z[:, None, :]]
    x1_fp = _interp_call(q1, s2, x2, x1, fp1, k_fp, n1)

    q0 = [px[..., None], py[..., None], pz[..., None]]
    F = _interp_call(q0, s1, x1_fp, None, fp0, k_fp, qb0)
    return F, g


def kernel(pts, params):
    return _encode(pts, params, n1=512, n2=128, r1=0.2, r2=0.4,
                   max_n1=32, max_n2=64, k_fp=3, qb1=128, qb0=512)


# speedup vs baseline: 1.2611x; 1.2611x over previous
import functools
import jax
import jax.numpy as jnp
from jax.experimental import pallas as pl
from jax.experimental.pallas import tpu as pltpu

_F32 = jnp.float32
_INF = float("inf")


def _fold_mlp(layers):
    out = []
    for layer in layers:
        W = layer["W"]
        if "gamma" in layer:
            W = W * (layer["gamma"] / jnp.sqrt(1.0 + 1e-5))[None, :]
            b = layer["beta"]
            relu = True
        else:
            b = layer["b"]
            relu = False
        out.append((W, b.reshape(1, -1), relu))
    return out


def _apply_tail(h, wb_refs, relus):
    for i, relu in enumerate(relus):
        h = h + wb_refs[2 * i + 1][...]
        if relu:
            h = jnp.maximum(h, 0.0)
        if i + 1 < len(relus):
            h = jnp.dot(h, wb_refs[2 * (i + 1)][...],
                        preferred_element_type=_F32)
    return h


def _fps_body(n_out, px_ref, py_ref, pz_ref, ox_ref, oy_ref, oz_ref, dist_ref):
    px = px_ref[...]
    py = py_ref[...]
    pz = pz_ref[...]
    B, N = px.shape
    iota_n = jax.lax.broadcasted_iota(jnp.int32, (B, N), 1)
    iota_o = jax.lax.broadcasted_iota(jnp.int32, (B, n_out), 1)
    dist_ref[...] = jnp.full((B, N), _INF, _F32)
    ox_ref[...] = jnp.zeros((B, n_out), _F32)
    oy_ref[...] = jnp.zeros((B, n_out), _F32)
    oz_ref[...] = jnp.zeros((B, n_out), _F32)

    def step(t, last):
        oh = iota_n == last
        sx = jnp.sum(jnp.where(oh, px, 0.0), axis=1, keepdims=True)
        sy = jnp.sum(jnp.where(oh, py, 0.0), axis=1, keepdims=True)
        sz = jnp.sum(jnp.where(oh, pz, 0.0), axis=1, keepdims=True)
        rec = iota_o == t
        ox_ref[...] += jnp.where(rec, sx, 0.0)
        oy_ref[...] += jnp.where(rec, sy, 0.0)
        oz_ref[...] += jnp.where(rec, sz, 0.0)
        d2 = (px - sx) ** 2 + (py - sy) ** 2 + (pz - sz) ** 2
        nd = jnp.minimum(dist_ref[...], d2)
        dist_ref[...] = nd
        m = jnp.max(nd, axis=1, keepdims=True)
        nxt = jnp.min(jnp.where(nd == m, iota_n, N), axis=1, keepdims=True)
        return nxt

    jax.lax.fori_loop(0, n_out, step, jnp.zeros((B, 1), jnp.int32))


def _fps_call(px, py, pz, n_out):
    B, N = px.shape
    out_sd = jax.ShapeDtypeStruct((B, n_out), _F32)
    return pl.pallas_call(
        functools.partial(_fps_body, n_out),
        out_shape=(out_sd, out_sd, out_sd),
        scratch_shapes=[pltpu.VMEM((B, N), _F32)],
    )(px, py, pz)


def _sa_body(qb, S, K, r2, relus_l, relus_g, nlayers_l, has_feat, *refs):
    qx = refs[0][0]
    qy = refs[1][0]
    qz = refs[2][0]
    px = refs[3][0]
    py = refs[4][0]
    pz = refs[5][0]
    i = 6
    feat = None
    if has_feat:
        feat = refs[i][0]
        i += 1
    wb = refs[i:i + 2 * (nlayers_l + len(relus_g))]
    o_ref = refs[-2]
    d2_ref = refs[-1]
    wb_l = wb[: 2 * nlayers_l]
    wb_g = wb[2 * nlayers_l:]

    d2 = (qx - px) ** 2 + (qy - py) ** 2 + (qz - pz) ** 2
    d2_ref[...] = jnp.where(d2 <= r2, d2, _INF)
    iota_s = jax.lax.broadcasted_iota(jnp.int32, (qb, S), 1)
    w1 = wb_l[0]
    c_feat = w1.shape[0] - 3
    cmid = wb_l[2 * (nlayers_l - 1)].shape[1]

    def rnd(t, acc):
        d2m = d2_ref[...]
        m = jnp.min(d2m, axis=1, keepdims=True)
        valid = m < _INF
        amin = jnp.min(jnp.where(d2m == m, iota_s, S), axis=1, keepdims=True)
        oh = iota_s == amin
        d2_ref[...] = jnp.where(oh, _INF, d2m)
        nx = jnp.sum(jnp.where(oh, px, 0.0), axis=1, keepdims=True)
        ny = jnp.sum(jnp.where(oh, py, 0.0), axis=1, keepdims=True)
        nz = jnp.sum(jnp.where(oh, pz, 0.0), axis=1, keepdims=True)
        if has_feat:
            xj = jnp.dot(oh.astype(_F32), feat, preferred_element_type=_F32)
            h = jnp.dot(xj, w1[0:c_feat, :], preferred_element_type=_F32)
        else:
            h = nx * w1[0:1, :] + ny * w1[1:2, :] + nz * w1[2:3, :]
        h = (h + (nx - qx) * w1[c_feat:c_feat + 1, :]
               + (ny - qy) * w1[c_feat + 1:c_feat + 2, :]
               + (nz - qz) * w1[c_feat + 2:c_feat + 3, :])
        h = _apply_tail(h, wb_l, relus_l)
        h = jnp.where(valid, h, -_INF)
        return jnp.maximum(acc, h)

    acc0 = jnp.full((qb, cmid), -_INF, _F32)
    acc = jax.lax.fori_loop(0, K, rnd, acc0)
    hg = jnp.dot(acc, wb_g[0][...], preferred_element_type=_F32)
    o_ref[0] = _apply_tail(hg, wb_g, relus_g)


def _sa_call(qpos, spos, feat, layers_l, layers_g, K, r2, qb):
    B, Nq, _ = qpos[0].shape
    S = spos[0].shape[2]
    relus_l = tuple(r for _, _, r in layers_l)
    relus_g = tuple(r for _, _, r in layers_g)
    cout = layers_g[-1][0].shape[1]
    grid = (B, Nq // qb)
    specs = []
    args = []
    for q in qpos:
        args.append(q)
        specs.append(pl.BlockSpec((1, qb, 1), lambda b, j: (b, j, 0)))
    for s in spos:
        args.append(s)
        specs.append(pl.BlockSpec((1, 1, S), lambda b, j: (b, 0, 0)))
    if feat is not None:
        C = feat.shape[2]
        args.append(feat)
        specs.append(pl.BlockSpec((1, S, C), lambda b, j: (b, 0, 0)))
    wbs = []
    for W, bias, _ in list(layers_l) + list(layers_g):
        wbs.extend([W, bias])
    for w in wbs:
        args.append(w)
        specs.append(pl.BlockSpec(w.shape, lambda b, j: (0, 0)))
    body = functools.partial(_sa_body, qb, S, K, r2, relus_l, relus_g,
                             len(layers_l), feat is not None)
    return pl.pallas_call(
        body,
        grid=grid,
        in_specs=specs,
        out_specs=pl.BlockSpec((1, qb, cout), lambda b, j: (b, j, 0)),
        out_shape=jax.ShapeDtypeStruct((B, Nq, cout), _F32),
        scratch_shapes=[pltpu.VMEM((qb, S), _F32)],
        compiler_params=pltpu.CompilerParams(
            dimension_semantics=("parallel", "parallel")),
    )(*args)


def _glob_body(relus, x_ref, w1_ref, b1_ref, w2_ref, b2_ref, o_ref):
    x = x_ref[0]
    mx = jnp.max(x, axis=0, keepdims=True)
    h = jnp.dot(mx, w1_ref[...], preferred_element_type=_F32)
    o_ref[0] = _apply_tail(h, (w1_ref, b1_ref, w2_ref, b2_ref), relus)


def _glob_call(x2, layers):
    B, Np, C = x2.shape
    relus = tuple(r for _, _, r in layers)
    cout = layers[-1][0].shape[1]
    (w1, b1, _), (w2, b2, _) = layers
    return pl.pallas_call(
        functools.partial(_glob_body, relus),
        grid=(B,),
        in_specs=[
            pl.BlockSpec((1, Np, C), lambda b: (b, 0, 0)),
            pl.BlockSpec(w1.shape, lambda b: (0, 0)),
            pl.BlockSpec(b1.shape, lambda b: (0, 0)),
            pl.BlockSpec(w2.shape, lambda b: (0, 0)),
            pl.BlockSpec(b2.shape, lambda b: (0, 0)),
        ],
        out_specs=pl.BlockSpec((1, 1, cout), lambda b: (b, 0, 0)),
        out_shape=jax.ShapeDtypeStruct((B, 1, cout), _F32),
        compiler_params=pltpu.CompilerParams(
            dimension_semantics=("parallel",)),
    )(x2, w1, b1, w2, b2)


def _interp_body(qb, S, k, relus, nskip, *refs):
    qx = refs[0][0]
    qy = refs[1][0]
    qz = refs[2][0]
    px = refs[3][0]
    py = refs[4][0]
    pz = refs[5][0]
    feat = refs[6][0]
    i = 7
    skip = None
    if nskip > 0:
        skip = refs[i][0]
        i += 1
    wb = refs[i:-1]
    o_ref = refs[-1]

    d2m = (qx - px) ** 2 + (qy - py) ** 2 + (qz - pz) ** 2
    iota_s = jax.lax.broadcasted_iota(jnp.int32, (qb, S), 1)
    C = feat.shape[1]
    num = jnp.zeros((qb, C), _F32)
    den = jnp.zeros((qb, 1), _F32)
    for _ in range(k):
        m = jnp.min(d2m, axis=1, keepdims=True)
        amin = jnp.min(jnp.where(d2m == m, iota_s, S), axis=1, keepdims=True)
        oh = iota_s == amin
        d2m = jnp.where(oh, _INF, d2m)
        w = 1.0 / jnp.maximum(m, 1e-16)
        ft = jnp.dot(oh.astype(_F32), feat, preferred_element_type=_F32)
        num = num + w * ft
        den = den + w
    up = num / den
    w1 = wb[0]
    h = jnp.dot(up, w1[0:C, :], preferred_element_type=_F32)
    if nskip > 0:
        cs = skip.shape[1]
        h = h + jnp.dot(skip, w1[C:C + cs, :], preferred_element_type=_F32)
    else:
        h = (h + qx * w1[C:C + 1, :] + qy * w1[C + 1:C + 2, :]
               + qz * w1[C + 2:C + 3, :])
    o_ref[0] = _apply_tail(h, wb, relus)


def _interp_call(qpos, spos, feat, skip, layers, k, qb):
    B, Nq, _ = qpos[0].shape
    S = feat.shape[1]
    relus = tuple(r for _, _, r in layers)
    cout = layers[-1][0].shape[1]
    C = feat.shape[2]
    grid = (B, Nq // qb)
    specs = []
    args = []
    for q in qpos:
        args.append(q)
        specs.append(pl.BlockSpec((1, qb, 1), lambda b, j: (b, j, 0)))
    for s in spos:
        args.append(s)
        specs.append(pl.BlockSpec((1, 1, S), lambda b, j: (b, 0, 0)))
    args.append(feat)
    specs.append(pl.BlockSpec((1, S, C), lambda b, j: (b, 0, 0)))
    nskip = 0
    if skip is not None:
        nskip = skip.shape[2]
        args.append(skip)
        specs.append(pl.BlockSpec((1, qb, nskip), lambda b, j: (b, j, 0)))
    for W, bias, _ in layers:
        args.append(W)
        specs.append(pl.BlockSpec(W.shape, lambda b, j: (0, 0)))
        args.append(bias)
        specs.append(pl.BlockSpec(bias.shape, lambda b, j: (0, 0)))
    body = functools.partial(_interp_body, qb, S, k, relus, nskip)
    return pl.pallas_call(
        body,
        grid=grid,
        in_specs=specs,
        out_specs=pl.BlockSpec((1, qb, cout), lambda b, j: (b, j, 0)),
        out_shape=jax.ShapeDtypeStruct((B, Nq, cout), _F32),
        compiler_params=pltpu.CompilerParams(
            dimension_semantics=("parallel", "parallel")),
    )(*args)


def _encode(pts, params, n1, n2, r1, r2, max_n1, max_n2, k_fp, qb1, qb0):
    B, N, _ = pts.shape
    px = pts[:, :, 0]
    py = pts[:, :, 1]
    pz = pts[:, :, 2]

    sa1_l = _fold_mlp(params["sa1_local"])
    sa1_g = _fold_mlp(params["sa1_global"])
    sa2_l = _fold_mlp(params["sa2_local"])
    sa2_g = _fold_mlp(params["sa2_global"])
    glob = _fold_mlp(params["glob"])
    fp1 = _fold_mlp(params["fp1"])
    fp0 = _fold_mlp(params["fp0"])

    c1x, c1y, c1z = _fps_call(px, py, pz, n1)
    q1 = [c1x[..., None], c1y[..., None], c1z[..., None]]
    s0 = [px[:, None, :], py[:, None, :], pz[:, None, :]]
    x1 = _sa_call(q1, s0, None, sa1_l, sa1_g, max_n1, r1 * r1, qb1)

    c2x, c2y, c2z = _fps_call(c1x, c1y, c1z, n2)
    q2 = [c2x[..., None], c2y[..., None], c2z[..., None]]
    s1 = [c1x[:, None, :], c1y[:, None, :], c1z[:, None, :]]
    x2 = _sa_call(q2, s1, x1, sa2_l, sa2_g, max_n2, r2 * r2, n2)

    g = _glob_call(x2, glob)[:, 0, :]

    s2 = [c2x[:, None, :], c2y[:, None, :], c2z[:, None, :]]
    x1_fp = _interp_call(q1, s2, x2, x1, fp1, k_fp, n1)

    q0 = [px[..., None], py[..., None], pz[..., None]]
    F = _interp_call(q0, s1, x1_fp, None, fp0, k_fp, qb0)
    return F, g


def kernel(pts, params):
    return _encode(pts, params, n1=512, n2=128, r1=0.2, r2=0.4,
                   max_n1=32, max_n2=64, k_fp=3, qb1=128, qb0=512)


# speedup vs baseline: 1.4139x; 1.1212x over previous
import functools
import jax
import jax.numpy as jnp
from jax.experimental import pallas as pl
from jax.experimental.pallas import tpu as pltpu

_F32 = jnp.float32
_INF = float("inf")


def _fold_mlp(layers):
    out = []
    for layer in layers:
        W = layer["W"]
        if "gamma" in layer:
            W = W * (layer["gamma"] / jnp.sqrt(1.0 + 1e-5))[None, :]
            b = layer["beta"]
            relu = True
        else:
            b = layer["b"]
            relu = False
        out.append((W, b.reshape(1, -1), relu))
    return out


def _apply_tail(h, wb_refs, relus):
    for i, relu in enumerate(relus):
        h = h + wb_refs[2 * i + 1][...]
        if relu:
            h = jnp.maximum(h, 0.0)
        if i + 1 < len(relus):
            h = jnp.dot(h, wb_refs[2 * (i + 1)][...],
                        preferred_element_type=_F32)
    return h


def _fps_body(n_out, px_ref, py_ref, pz_ref, ox_ref, oy_ref, oz_ref, dist_ref):
    px = px_ref[...]
    py = py_ref[...]
    pz = pz_ref[...]
    B, N = px.shape
    iota_n = jax.lax.broadcasted_iota(jnp.int32, (B, N), 1)
    iota_o = jax.lax.broadcasted_iota(jnp.int32, (B, n_out), 1)
    dist_ref[...] = jnp.full((B, N), _INF, _F32)
    ox_ref[...] = jnp.zeros((B, n_out), _F32)
    oy_ref[...] = jnp.zeros((B, n_out), _F32)
    oz_ref[...] = jnp.zeros((B, n_out), _F32)

    def step(t, last):
        oh = iota_n == last
        sx = jnp.sum(jnp.where(oh, px, 0.0), axis=1, keepdims=True)
        sy = jnp.sum(jnp.where(oh, py, 0.0), axis=1, keepdims=True)
        sz = jnp.sum(jnp.where(oh, pz, 0.0), axis=1, keepdims=True)
        rec = iota_o == t
        ox_ref[...] += jnp.where(rec, sx, 0.0)
        oy_ref[...] += jnp.where(rec, sy, 0.0)
        oz_ref[...] += jnp.where(rec, sz, 0.0)
        d2 = (px - sx) ** 2 + (py - sy) ** 2 + (pz - sz) ** 2
        nd = jnp.minimum(dist_ref[...], d2)
        dist_ref[...] = nd
        m = jnp.max(nd, axis=1, keepdims=True)
        nxt = jnp.min(jnp.where(nd == m, iota_n, N), axis=1, keepdims=True)
        return nxt

    jax.lax.fori_loop(0, n_out, step, jnp.zeros((B, 1), jnp.int32))


def _fps_call(px, py, pz, n_out):
    B, N = px.shape
    out_sd = jax.ShapeDtypeStruct((B, n_out), _F32)
    return pl.pallas_call(
        functools.partial(_fps_body, n_out),
        out_shape=(out_sd, out_sd, out_sd),
        scratch_shapes=[pltpu.VMEM((B, N), _F32)],
    )(px, py, pz)


def _sa_body(qb, S, K, r2, relus_l, relus_g, nlayers_l, has_feat, *refs):
    qx = refs[0][0]
    qy = refs[1][0]
    qz = refs[2][0]
    px = refs[3][0]
    py = refs[4][0]
    pz = refs[5][0]
    i = 6
    feat = None
    if has_feat:
        feat = refs[i][0]
        i += 1
    wb = refs[i:i + 2 * (nlayers_l + len(relus_g))]
    o_ref = refs[-2]
    d2_ref = refs[-1]
    wb_l = wb[: 2 * nlayers_l]
    wb_g = wb[2 * nlayers_l:]

    d2 = (qx - px) ** 2 + (qy - py) ** 2 + (qz - pz) ** 2
    in_r = d2 <= r2
    d2_ref[...] = jnp.where(in_r, d2, _INF)
    cnt = jnp.sum(in_r.astype(jnp.int32), axis=1)
    nrounds = jnp.minimum(K, jnp.max(cnt))
    w1 = wb_l[0]
    c_feat = w1.shape[0] - 3
    cmid = wb_l[2 * (nlayers_l - 1)].shape[1]

    def rnd(t, acc):
        iota_s = jax.lax.broadcasted_iota(jnp.int32, (qb, S), 1)
        d2m = d2_ref[...]
        m = jnp.min(d2m, axis=1, keepdims=True)
        valid = m < _INF
        amin = jnp.min(jnp.where(d2m == m, iota_s, S), axis=1, keepdims=True)
        oh = iota_s == amin
        d2_ref[...] = jnp.where(oh, _INF, d2m)
        nx = jnp.sum(jnp.where(oh, px, 0.0), axis=1, keepdims=True)
        ny = jnp.sum(jnp.where(oh, py, 0.0), axis=1, keepdims=True)
        nz = jnp.sum(jnp.where(oh, pz, 0.0), axis=1, keepdims=True)
        if has_feat:
            xj = jnp.dot(oh.astype(_F32), feat, preferred_element_type=_F32)
            h = jnp.dot(xj, w1[0:c_feat, :], preferred_element_type=_F32)
        else:
            h = nx * w1[0:1, :] + ny * w1[1:2, :] + nz * w1[2:3, :]
        h = (h + (nx - qx) * w1[c_feat:c_feat + 1, :]
               + (ny - qy) * w1[c_feat + 1:c_feat + 2, :]
               + (nz - qz) * w1[c_feat + 2:c_feat + 3, :])
        h = _apply_tail(h, wb_l, relus_l)
        h = jnp.where(valid, h, -_INF)
        return jnp.maximum(acc, h)

    acc0 = jnp.full((qb, cmid), -_INF, _F32)
    acc = jax.lax.fori_loop(0, nrounds, rnd, acc0)
    hg = jnp.dot(acc, wb_g[0][...], preferred_element_type=_F32)
    o_ref[0] = _apply_tail(hg, wb_g, relus_g)


def _sa_call(qpos, spos, feat, layers_l, layers_g, K, r2, qb):
    B, Nq, _ = qpos[0].shape
    S = spos[0].shape[2]
    relus_l = tuple(r for _, _, r in layers_l)
    relus_g = tuple(r for _, _, r in layers_g)
    cout = layers_g[-1][0].shape[1]
    grid = (B, Nq // qb)
    specs = []
    args = []
    for q in qpos:
        args.append(q)
        specs.append(pl.BlockSpec((1, qb, 1), lambda b, j: (b, j, 0)))
    for s in spos:
        args.append(s)
        specs.append(pl.BlockSpec((1, 1, S), lambda b, j: (b, 0, 0)))
    if feat is not None:
        C = feat.shape[2]
        args.append(feat)
        specs.append(pl.BlockSpec((1, S, C), lambda b, j: (b, 0, 0)))
    wbs = []
    for W, bias, _ in list(layers_l) + list(layers_g):
        wbs.extend([W, bias])
    for w in wbs:
        args.append(w)
        specs.append(pl.BlockSpec(w.shape, lambda b, j: (0, 0)))
    body = functools.partial(_sa_body, qb, S, K, r2, relus_l, relus_g,
                             len(layers_l), feat is not None)
    return pl.pallas_call(
        body,
        grid=grid,
        in_specs=specs,
        out_specs=pl.BlockSpec((1, qb, cout), lambda b, j: (b, j, 0)),
        out_shape=jax.ShapeDtypeStruct((B, Nq, cout), _F32),
        scratch_shapes=[pltpu.VMEM((qb, S), _F32)],
        compiler_params=pltpu.CompilerParams(
            dimension_semantics=("parallel", "parallel")),
    )(*args)


def _glob_body(relus, x_ref, w1_ref, b1_ref, w2_ref, b2_ref, o_ref):
    x = x_ref[0]
    mx = jnp.max(x, axis=0, keepdims=True)
    h = jnp.dot(mx, w1_ref[...], preferred_element_type=_F32)
    o_ref[0] = _apply_tail(h, (w1_ref, b1_ref, w2_ref, b2_ref), relus)


def _glob_call(x2, layers):
    B, Np, C = x2.shape
    relus = tuple(r for _, _, r in layers)
    cout = layers[-1][0].shape[1]
    (w1, b1, _), (w2, b2, _) = layers
    return pl.pallas_call(
        functools.partial(_glob_body, relus),
        grid=(B,),
        in_specs=[
            pl.BlockSpec((1, Np, C), lambda b: (b, 0, 0)),
            pl.BlockSpec(w1.shape, lambda b: (0, 0)),
            pl.BlockSpec(b1.shape, lambda b: (0, 0)),
            pl.BlockSpec(w2.shape, lambda b: (0, 0)),
            pl.BlockSpec(b2.shape, lambda b: (0, 0)),
        ],
        out_specs=pl.BlockSpec((1, 1, cout), lambda b: (b, 0, 0)),
        out_shape=jax.ShapeDtypeStruct((B, 1, cout), _F32),
        compiler_params=pltpu.CompilerParams(
            dimension_semantics=("parallel",)),
    )(x2, w1, b1, w2, b2)


def _interp_body(qb, S, k, relus, nskip, *refs):
    qx = refs[0][0]
    qy = refs[1][0]
    qz = refs[2][0]
    px = refs[3][0]
    py = refs[4][0]
    pz = refs[5][0]
    feat = refs[6][0]
    i = 7
    skip = None
    if nskip > 0:
        skip = refs[i][0]
        i += 1
    wb = refs[i:-1]
    o_ref = refs[-1]

    d2m = (qx - px) ** 2 + (qy - py) ** 2 + (qz - pz) ** 2
    iota_s = jax.lax.broadcasted_iota(jnp.int32, (qb, S), 1)
    C = feat.shape[1]
    num = jnp.zeros((qb, C), _F32)
    den = jnp.zeros((qb, 1), _F32)
    for _ in range(k):
        m = jnp.min(d2m, axis=1, keepdims=True)
        amin = jnp.min(jnp.where(d2m == m, iota_s, S), axis=1, keepdims=True)
        oh = iota_s == amin
        d2m = jnp.where(oh, _INF, d2m)
        w = 1.0 / jnp.maximum(m, 1e-16)
        ft = jnp.dot(oh.astype(_F32), feat, preferred_element_type=_F32)
        num = num + w * ft
        den = den + w
    up = num / den
    w1 = wb[0]
    h = jnp.dot(up, w1[0:C, :], preferred_element_type=_F32)
    if nskip > 0:
        cs = skip.shape[1]
        h = h + jnp.dot(skip, w1[C:C + cs, :], preferred_element_type=_F32)
    else:
        h = (h + qx * w1[C:C + 1, :] + qy * w1[C + 1:C + 2, :]
               + qz * w1[C + 2:C + 3, :])
    o_ref[0] = _apply_tail(h, wb, relus)


def _interp_call(qpos, spos, feat, skip, layers, k, qb):
    B, Nq, _ = qpos[0].shape
    S = feat.shape[1]
    relus = tuple(r for _, _, r in layers)
    cout = layers[-1][0].shape[1]
    C = feat.shape[2]
    grid = (B, Nq // qb)
    specs = []
    args = []
    for q in qpos:
        args.append(q)
        specs.append(pl.BlockSpec((1, qb, 1), lambda b, j: (b, j, 0)))
    for s in spos:
        args.append(s)
        specs.append(pl.BlockSpec((1, 1, S), lambda b, j: (b, 0, 0)))
    args.append(feat)
    specs.append(pl.BlockSpec((1, S, C), lambda b, j: (b, 0, 0)))
    nskip = 0
    if skip is not None:
        nskip = skip.shape[2]
        args.append(skip)
        specs.append(pl.BlockSpec((1, qb, nskip), lambda b, j: (b, j, 0)))
    for W, bias, _ in layers:
        args.append(W)
        specs.append(pl.BlockSpec(W.shape, lambda b, j: (0, 0)))
        args.append(bias)
        specs.append(pl.BlockSpec(bias.shape, lambda b, j: (0, 0)))
    body = functools.partial(_interp_body, qb, S, k, relus, nskip)
    return pl.pallas_call(
        body,
        grid=grid,
        in_specs=specs,
        out_specs=pl.BlockSpec((1, qb, cout), lambda b, j: (b, j, 0)),
        out_shape=jax.ShapeDtypeStruct((B, Nq, cout), _F32),
        compiler_params=pltpu.CompilerParams(
            dimension_semantics=("parallel", "parallel")),
    )(*args)


def _encode(pts, params, n1, n2, r1, r2, max_n1, max_n2, k_fp, qb1, qb0):
    B, N, _ = pts.shape
    px = pts[:, :, 0]
    py = pts[:, :, 1]
    pz = pts[:, :, 2]

    sa1_l = _fold_mlp(params["sa1_local"])
    sa1_g = _fold_mlp(params["sa1_global"])
    sa2_l = _fold_mlp(params["sa2_local"])
    sa2_g = _fold_mlp(params["sa2_global"])
    glob = _fold_mlp(params["glob"])
    fp1 = _fold_mlp(params["fp1"])
    fp0 = _fold_mlp(params["fp0"])

    c1x, c1y, c1z = _fps_call(px, py, pz, n1)
    q1 = [c1x[..., None], c1y[..., None], c1z[..., None]]
    s0 = [px[:, None, :], py[:, None, :], pz[:, None, :]]
    x1 = _sa_call(q1, s0, None, sa1_l, sa1_g, max_n1, r1 * r1, qb1)

    c2x, c2y, c2z = _fps_call(c1x, c1y, c1z, n2)
    q2 = [c2x[..., None], c2y[..., None], c2z[..., None]]
    s1 = [c1x[:, None, :], c1y[:, None, :], c1z[:, None, :]]
    x2 = _sa_call(q2, s1, x1, sa2_l, sa2_g, max_n2, r2 * r2, n2)

    g = _glob_call(x2, glob)[:, 0, :]

    s2 = [c2x[:, None, :], c2y[:, None, :], c2z[:, None, :]]
    x1_fp = _interp_call(q1, s2, x2, x1, fp1, k_fp, n1)

    q0 = [px[..., None], py[..., None], pz[..., None]]
    F = _interp_call(q0, s1, x1_fp, None, fp0, k_fp, qb0)
    return F, g


def kernel(pts, params):
    return _encode(pts, params, n1=512, n2=128, r1=0.2, r2=0.4,
                   max_n1=32, max_n2=64, k_fp=3, qb1=128, qb0=512)


# speedup vs baseline: 2.6881x; 1.9012x over previous
import functools
import jax
import jax.numpy as jnp
from jax.experimental import pallas as pl
from jax.experimental.pallas import tpu as pltpu

_F32 = jnp.float32
_INF = float("inf")


def _fold_mlp(layers):
    out = []
    for layer in layers:
        W = layer["W"]
        if "gamma" in layer:
            W = W * (layer["gamma"] / jnp.sqrt(1.0 + 1e-5))[None, :]
            b = layer["beta"]
            relu = True
        else:
            b = layer["b"]
            relu = False
        out.append((W, b.reshape(1, -1), relu))
    return out


def _apply_tail(h, wb_refs, relus):
    for i, relu in enumerate(relus):
        h = h + wb_refs[2 * i + 1][...]
        if relu:
            h = jnp.maximum(h, 0.0)
        if i + 1 < len(relus):
            h = jnp.dot(h, wb_refs[2 * (i + 1)][...],
                        preferred_element_type=_F32)
    return h


def _fps_body(n_out, px_ref, py_ref, pz_ref, ox_ref, oy_ref, oz_ref, dist_ref):
    px = px_ref[...]
    py = py_ref[...]
    pz = pz_ref[...]
    B, N = px.shape
    iota_n = jax.lax.broadcasted_iota(jnp.int32, (B, N), 1)
    iota_o = jax.lax.broadcasted_iota(jnp.int32, (B, n_out), 1)
    dist_ref[...] = jnp.full((B, N), _INF, _F32)
    ox_ref[...] = jnp.zeros((B, n_out), _F32)
    oy_ref[...] = jnp.zeros((B, n_out), _F32)
    oz_ref[...] = jnp.zeros((B, n_out), _F32)

    def step(t, last):
        oh = iota_n == last
        sx = jnp.sum(jnp.where(oh, px, 0.0), axis=1, keepdims=True)
        sy = jnp.sum(jnp.where(oh, py, 0.0), axis=1, keepdims=True)
        sz = jnp.sum(jnp.where(oh, pz, 0.0), axis=1, keepdims=True)
        rec = iota_o == t
        ox_ref[...] += jnp.where(rec, sx, 0.0)
        oy_ref[...] += jnp.where(rec, sy, 0.0)
        oz_ref[...] += jnp.where(rec, sz, 0.0)
        d2 = (px - sx) ** 2 + (py - sy) ** 2 + (pz - sz) ** 2
        nd = jnp.minimum(dist_ref[...], d2)
        dist_ref[...] = nd
        m = jnp.max(nd, axis=1, keepdims=True)
        nxt = jnp.min(jnp.where(nd == m, iota_n, N), axis=1, keepdims=True)
        return nxt

    jax.lax.fori_loop(0, n_out, step, jnp.zeros((B, 1), jnp.int32))


def _fps_call(px, py, pz, n_out):
    B, N = px.shape
    out_sd = jax.ShapeDtypeStruct((B, n_out), _F32)
    return pl.pallas_call(
        functools.partial(_fps_body, n_out),
        out_shape=(out_sd, out_sd, out_sd),
        scratch_shapes=[pltpu.VMEM((B, N), _F32)],
    )(px, py, pz)


def _sa_body(qb, S, K, r2, relus_l, relus_g, nlayers_l, has_feat, nch, *refs):
    qx = refs[0][0]
    qy = refs[1][0]
    qz = refs[2][0]
    px = refs[3][0]
    py = refs[4][0]
    pz = refs[5][0]
    i = 6
    feat = None
    if has_feat:
        feat = refs[i][0]
        i += 1
    wb = refs[i:i + 2 * (nlayers_l + len(relus_g))]
    i += 2 * (nlayers_l + len(relus_g))
    o_ref = refs[i]
    d2_ref = refs[i + 1]
    two = (not has_feat) and nch > 1
    if two:
        cd_ref, cx_ref, cy_ref, cz_ref = refs[i + 2:i + 6]
    wb_l = wb[: 2 * nlayers_l]
    wb_g = wb[2 * nlayers_l:]

    d2 = (qx - px) ** 2 + (qy - py) ** 2 + (qz - pz) ** 2
    in_r = d2 <= r2
    d2_ref[...] = jnp.where(in_r, d2, _INF)
    cnt = jnp.sum(in_r.astype(jnp.int32), axis=1)
    nrounds = jnp.minimum(K, jnp.max(cnt))
    w1 = wb_l[0]
    c_feat = w1.shape[0] - 3
    cmid = wb_l[2 * (nlayers_l - 1)].shape[1]

    def mlp_point(nx, ny, nz, xj, valid, acc):
        if xj is not None:
            h = jnp.dot(xj, w1[0:c_feat, :], preferred_element_type=_F32)
        else:
            h = nx * w1[0:1, :] + ny * w1[1:2, :] + nz * w1[2:3, :]
        h = (h + (nx - qx) * w1[c_feat:c_feat + 1, :]
               + (ny - qy) * w1[c_feat + 1:c_feat + 2, :]
               + (nz - qz) * w1[c_feat + 2:c_feat + 3, :])
        h = _apply_tail(h, wb_l, relus_l)
        h = jnp.where(valid, h, -_INF)
        return jnp.maximum(acc, h)

    acc0 = jnp.full((qb, cmid), -_INF, _F32)

    if two:
        NCOL = nch * K
        CS = S // nch
        cd_ref[...] = jnp.full((qb, NCOL), _INF, _F32)
        cx_ref[...] = jnp.zeros((qb, NCOL), _F32)
        cy_ref[...] = jnp.zeros((qb, NCOL), _F32)
        cz_ref[...] = jnp.zeros((qb, NCOL), _F32)
        iota_col = jax.lax.broadcasted_iota(jnp.int32, (1, NCOL), 1)
        for c in range(nch):
            sl = slice(c * CS, (c + 1) * CS)
            cntc = jnp.sum((d2_ref[:, sl] < _INF).astype(jnp.int32), axis=1)
            rc = jnp.minimum(K, jnp.max(cntc))
            pxc = px[:, sl]
            pyc = py[:, sl]
            pzc = pz[:, sl]

            def chunk_rnd(t, _, c=c, sl=sl, pxc=pxc, pyc=pyc, pzc=pzc):
                iota_cs = jax.lax.broadcasted_iota(jnp.int32, (qb, CS), 1)
                d2m = d2_ref[:, sl]
                m = jnp.min(d2m, axis=1, keepdims=True)
                amin = jnp.min(jnp.where(d2m == m, iota_cs, CS), axis=1,
                               keepdims=True)
                oh = iota_cs == amin
                d2_ref[:, sl] = jnp.where(oh, _INF, d2m)
                nx = jnp.sum(jnp.where(oh, pxc, 0.0), axis=1, keepdims=True)
                ny = jnp.sum(jnp.where(oh, pyc, 0.0), axis=1, keepdims=True)
                nz = jnp.sum(jnp.where(oh, pzc, 0.0), axis=1, keepdims=True)
                rec = iota_col == (c * K + t)
                cd_ref[...] = jnp.where(rec, m, cd_ref[...])
                cx_ref[...] = jnp.where(rec, nx, cx_ref[...])
                cy_ref[...] = jnp.where(rec, ny, cy_ref[...])
                cz_ref[...] = jnp.where(rec, nz, cz_ref[...])
                return 0

            jax.lax.fori_loop(0, rc, chunk_rnd, 0)

        def rnd2(t, acc):
            iota_c = jax.lax.broadcasted_iota(jnp.int32, (qb, NCOL), 1)
            d2m = cd_ref[...]
            m = jnp.min(d2m, axis=1, keepdims=True)
            valid = m < _INF
            amin = jnp.min(jnp.where(d2m == m, iota_c, NCOL), axis=1,
                           keepdims=True)
            oh = iota_c == amin
            cd_ref[...] = jnp.where(oh, _INF, d2m)
            nx = jnp.sum(jnp.where(oh, cx_ref[...], 0.0), axis=1,
                         keepdims=True)
            ny = jnp.sum(jnp.where(oh, cy_ref[...], 0.0), axis=1,
                         keepdims=True)
            nz = jnp.sum(jnp.where(oh, cz_ref[...], 0.0), axis=1,
                         keepdims=True)
            return mlp_point(nx, ny, nz, None, valid, acc)

        acc = jax.lax.fori_loop(0, nrounds, rnd2, acc0)
    else:
        def rnd(t, acc):
            iota_s = jax.lax.broadcasted_iota(jnp.int32, (qb, S), 1)
            d2m = d2_ref[...]
            m = jnp.min(d2m, axis=1, keepdims=True)
            valid = m < _INF
            amin = jnp.min(jnp.where(d2m == m, iota_s, S), axis=1,
                           keepdims=True)
            oh = iota_s == amin
            d2_ref[...] = jnp.where(oh, _INF, d2m)
            nx = jnp.sum(jnp.where(oh, px, 0.0), axis=1, keepdims=True)
            ny = jnp.sum(jnp.where(oh, py, 0.0), axis=1, keepdims=True)
            nz = jnp.sum(jnp.where(oh, pz, 0.0), axis=1, keepdims=True)
            xj = None
            if has_feat:
                xj = jnp.dot(oh.astype(_F32), feat,
                             preferred_element_type=_F32)
            return mlp_point(nx, ny, nz, xj, valid, acc)

        acc = jax.lax.fori_loop(0, nrounds, rnd, acc0)

    hg = jnp.dot(acc, wb_g[0][...], preferred_element_type=_F32)
    o_ref[0] = _apply_tail(hg, wb_g, relus_g)


def _sa_call(qpos, spos, feat, layers_l, layers_g, K, r2, qb, nch=1):
    B, Nq, _ = qpos[0].shape
    S = spos[0].shape[2]
    relus_l = tuple(r for _, _, r in layers_l)
    relus_g = tuple(r for _, _, r in layers_g)
    cout = layers_g[-1][0].shape[1]
    grid = (B, Nq // qb)
    specs = []
    args = []
    for q in qpos:
        args.append(q)
        specs.append(pl.BlockSpec((1, qb, 1), lambda b, j: (b, j, 0)))
    for s in spos:
        args.append(s)
        specs.append(pl.BlockSpec((1, 1, S), lambda b, j: (b, 0, 0)))
    if feat is not None:
        C = feat.shape[2]
        args.append(feat)
        specs.append(pl.BlockSpec((1, S, C), lambda b, j: (b, 0, 0)))
    wbs = []
    for W, bias, _ in list(layers_l) + list(layers_g):
        wbs.extend([W, bias])
    for w in wbs:
        args.append(w)
        specs.append(pl.BlockSpec(w.shape, lambda b, j: (0, 0)))
    body = functools.partial(_sa_body, qb, S, K, r2, relus_l, relus_g,
                             len(layers_l), feat is not None, nch)
    scratch = [pltpu.VMEM((qb, S), _F32)]
    if feat is None and nch > 1:
        scratch += [pltpu.VMEM((qb, nch * K), _F32) for _ in range(4)]
    return pl.pallas_call(
        body,
        grid=grid,
        in_specs=specs,
        out_specs=pl.BlockSpec((1, qb, cout), lambda b, j: (b, j, 0)),
        out_shape=jax.ShapeDtypeStruct((B, Nq, cout), _F32),
        scratch_shapes=scratch,
        compiler_params=pltpu.CompilerParams(
            dimension_semantics=("parallel", "parallel")),
    )(*args)


def _glob_body(relus, x_ref, w1_ref, b1_ref, w2_ref, b2_ref, o_ref):
    x = x_ref[0]
    mx = jnp.max(x, axis=0, keepdims=True)
    h = jnp.dot(mx, w1_ref[...], preferred_element_type=_F32)
    o_ref[0] = _apply_tail(h, (w1_ref, b1_ref, w2_ref, b2_ref), relus)


def _glob_call(x2, layers):
    B, Np, C = x2.shape
    relus = tuple(r for _, _, r in layers)
    cout = layers[-1][0].shape[1]
    (w1, b1, _), (w2, b2, _) = layers
    return pl.pallas_call(
        functools.partial(_glob_body, relus),
        grid=(B,),
        in_specs=[
            pl.BlockSpec((1, Np, C), lambda b: (b, 0, 0)),
            pl.BlockSpec(w1.shape, lambda b: (0, 0)),
            pl.BlockSpec(b1.shape, lambda b: (0, 0)),
            pl.BlockSpec(w2.shape, lambda b: (0, 0)),
            pl.BlockSpec(b2.shape, lambda b: (0, 0)),
        ],
        out_specs=pl.BlockSpec((1, 1, cout), lambda b: (b, 0, 0)),
        out_shape=jax.ShapeDtypeStruct((B, 1, cout), _F32),
        compiler_params=pltpu.CompilerParams(
            dimension_semantics=("parallel",)),
    )(x2, w1, b1, w2, b2)


def _interp_body(qb, S, k, relus, nskip, *refs):
    qx = refs[0][0]
    qy = refs[1][0]
    qz = refs[2][0]
    px = refs[3][0]
    py = refs[4][0]
    pz = refs[5][0]
    feat = refs[6][0]
    i = 7
    skip = None
    if nskip > 0:
        skip = refs[i][0]
        i += 1
    wb = refs[i:-1]
    o_ref = refs[-1]

    d2m = (qx - px) ** 2 + (qy - py) ** 2 + (qz - pz) ** 2
    iota_s = jax.lax.broadcasted_iota(jnp.int32, (qb, S), 1)
    C = feat.shape[1]
    num = jnp.zeros((qb, C), _F32)
    den = jnp.zeros((qb, 1), _F32)
    for _ in range(k):
        m = jnp.min(d2m, axis=1, keepdims=True)
        amin = jnp.min(jnp.where(d2m == m, iota_s, S), axis=1, keepdims=True)
        oh = iota_s == amin
        d2m = jnp.where(oh, _INF, d2m)
        w = 1.0 / jnp.maximum(m, 1e-16)
        ft = jnp.dot(oh.astype(_F32), feat, preferred_element_type=_F32)
        num = num + w * ft
        den = den + w
    up = num / den
    w1 = wb[0]
    h = jnp.dot(up, w1[0:C, :], preferred_element_type=_F32)
    if nskip > 0:
        cs = skip.shape[1]
        h = h + jnp.dot(skip, w1[C:C + cs, :], preferred_element_type=_F32)
    else:
        h = (h + qx * w1[C:C + 1, :] + qy * w1[C + 1:C + 2, :]
               + qz * w1[C + 2:C + 3, :])
    o_ref[0] = _apply_tail(h, wb, relus)


def _interp_call(qpos, spos, feat, skip, layers, k, qb):
    B, Nq, _ = qpos[0].shape
    S = feat.shape[1]
    relus = tuple(r for _, _, r in layers)
    cout = layers[-1][0].shape[1]
    C = feat.shape[2]
    grid = (B, Nq // qb)
    specs = []
    args = []
    for q in qpos:
        args.append(q)
        specs.append(pl.BlockSpec((1, qb, 1), lambda b, j: (b, j, 0)))
    for s in spos:
        args.append(s)
        specs.append(pl.BlockSpec((1, 1, S), lambda b, j: (b, 0, 0)))
    args.append(feat)
    specs.append(pl.BlockSpec((1, S, C), lambda b, j: (b, 0, 0)))
    nskip = 0
    if skip is not None:
        nskip = skip.shape[2]
        args.append(skip)
        specs.append(pl.BlockSpec((1, qb, nskip), lambda b, j: (b, j, 0)))
    for W, bias, _ in layers:
        args.append(W)
        specs.append(pl.BlockSpec(W.shape, lambda b, j: (0, 0)))
        args.append(bias)
        specs.append(pl.BlockSpec(bias.shape, lambda b, j: (0, 0)))
    body = functools.partial(_interp_body, qb, S, k, relus, nskip)
    return pl.pallas_call(
        body,
        grid=grid,
        in_specs=specs,
        out_specs=pl.BlockSpec((1, qb, cout), lambda b, j: (b, j, 0)),
        out_shape=jax.ShapeDtypeStruct((B, Nq, cout), _F32),
        compiler_params=pltpu.CompilerParams(
            dimension_semantics=("parallel", "parallel")),
    )(*args)


def _encode(pts, params, n1, n2, r1, r2, max_n1, max_n2, k_fp, qb1, qb0):
    B, N, _ = pts.shape
    px = pts[:, :, 0]
    py = pts[:, :, 1]
    pz = pts[:, :, 2]

    sa1_l = _fold_mlp(params["sa1_local"])
    sa1_g = _fold_mlp(params["sa1_global"])
    sa2_l = _fold_mlp(params["sa2_local"])
    sa2_g = _fold_mlp(params["sa2_global"])
    glob = _fold_mlp(params["glob"])
    fp1 = _fold_mlp(params["fp1"])
    fp0 = _fold_mlp(params["fp0"])

    c1x, c1y, c1z = _fps_call(px, py, pz, n1)
    q1 = [c1x[..., None], c1y[..., None], c1z[..., None]]
    s0 = [px[:, None, :], py[:, None, :], pz[:, None, :]]
    x1 = _sa_call(q1, s0, None, sa1_l, sa1_g, max_n1, r1 * r1, qb1, nch=8)

    c2x, c2y, c2z = _fps_call(c1x, c1y, c1z, n2)
    q2 = [c2x[..., None], c2y[..., None], c2z[..., None]]
    s1 = [c1x[:, None, :], c1y[:, None, :], c1z[:, None, :]]
    x2 = _sa_call(q2, s1, x1, sa2_l, sa2_g, max_n2, r2 * r2, n2)

    g = _glob_call(x2, glob)[:, 0, :]

    s2 = [c2x[:, None, :], c2y[:, None, :], c2z[:, None, :]]
    x1_fp = _interp_call(q1, s2, x2, x1, fp1, k_fp, n1)

    q0 = [px[..., None], py[..., None], pz[..., None]]
    F = _interp_call(q0, s1, x1_fp, None, fp0, k_fp, qb0)
    return F, g


def kernel(pts, params):
    return _encode(pts, params, n1=512, n2=128, r1=0.2, r2=0.4,
                   max_n1=32, max_n2=64, k_fp=3, qb1=128, qb0=512)
